# baseline (speedup 1.0000x reference)
import functools

import jax
import jax.numpy as jnp
from jax.experimental import pallas as pl
from jax.experimental.pallas import tpu as pltpu

_W = 128


def _unary_kernel(x_ref, out_ref, *, L):
    j = pl.program_id(1)
    x = x_ref[...]
    bB = x.shape[0]
    col = jax.lax.broadcasted_iota(jnp.int32, (bB, L, _W), 2) + j * _W
    row = jax.lax.broadcasted_iota(jnp.int32, (bB, L, _W), 1)
    out_ref[...] = ((col == row) | (col - L == x[:, :, None])).astype(jnp.float32)


def _binary_kernel(x_ref, xk0_ref, xk1_ref, out_ref, *, L):
    j = pl.program_id(1)
    x = x_ref[...]
    bB = x.shape[0]
    ik_row = jax.lax.broadcasted_iota(jnp.int32, (bB, L, _W), 1)
    ik_col = jax.lax.broadcasted_iota(jnp.int32, (bB, L, _W), 2) + j * _W
    xk0 = xk0_ref[...][:, None, :]
    xk1 = xk1_ref[...][:, None, :]
    xj = jnp.where(ik_col < ik_row, xk0, xk1)
    out_ref[...] = (x[:, :, None] == xj).astype(jnp.float32)


@functools.partial(jax.jit, static_argnames=("bB",))
def _run(inputs, bB):
    B, L = inputs.shape
    S1 = 101
    C = L + S1
    params = pltpu.CompilerParams(dimension_semantics=("parallel", "arbitrary"))

    unary = pl.pallas_call(
        functools.partial(_unary_kernel, L=L),
        grid=(B // bB, pl.cdiv(C, _W)),
        in_specs=[pl.BlockSpec((bB, L), lambda b, j: (b, 0))],
        out_specs=pl.BlockSpec((bB, L, _W), lambda b, j: (b, 0, j)),
        out_shape=jax.ShapeDtypeStruct((B, L, C), jnp.float32),
        compiler_params=params,
    )(inputs)

    nj = pl.cdiv(L - 1, _W)
    P = (nj + 1) * _W
    xp = jnp.pad(inputs, ((0, 0), (0, P - L)))
    xs = jnp.pad(inputs[:, 1:], ((0, 0), (0, P - L + 1)))
    binary = pl.pallas_call(
        functools.partial(_binary_kernel, L=L),
        grid=(B // bB, nj),
        in_specs=[
            pl.BlockSpec((bB, L), lambda b, j: (b, 0)),
            pl.BlockSpec((bB, _W), lambda b, j: (b, j)),
            pl.BlockSpec((bB, _W), lambda b, j: (b, j)),
        ],
        out_specs=pl.BlockSpec((bB, L, _W), lambda b, j: (b, 0, j)),
        out_shape=jax.ShapeDtypeStruct((B, L, L - 1), jnp.float32),
        compiler_params=params,
    )(inputs, xp, xs)
    return unary, binary[..., None]


def kernel(inputs, table):
    del table
    unary, binary = _run(inputs, bB=32)
    return (unary, binary)


# speedup vs baseline: 2.3400x; 2.3400x over previous
import functools

import jax
import jax.numpy as jnp
from jax.experimental import pallas as pl
from jax.experimental.pallas import tpu as pltpu


def _unary_kernel(xt_ref, out_ref, *, L, Cb):
    c0 = pl.program_id(0) * Cb
    xt = xt_ref[...]
    Bb = xt.shape[1]
    ci = jax.lax.broadcasted_iota(jnp.int32, (Cb, L, Bb), 0) + c0
    ii = jax.lax.broadcasted_iota(jnp.int32, (Cb, L, Bb), 1)
    out_ref[...] = ((ci == ii) | (ci - L == xt[None, :, :])).astype(jnp.float32)


def _binary_kernel(xi_ref, xt_ref, out_ref, *, L, Ib):
    i0 = pl.program_id(0) * Ib
    xi = xi_ref[...]
    xt = xt_ref[...]
    Bb = xt.shape[1]
    kk = jax.lax.broadcasted_iota(jnp.int32, (Ib, L - 1, Bb), 1)
    ii = jax.lax.broadcasted_iota(jnp.int32, (Ib, L - 1, Bb), 0) + i0
    xk0 = xt[None, : L - 1, :]
    xk1 = xt[None, 1:L, :]
    xj = jnp.where(kk < ii, xk0, xk1)
    out_ref[...] = (xi[:, None, :] == xj).astype(jnp.float32)


@jax.jit
def _run(inputs):
    B, L = inputs.shape
    S1 = 101
    C = L + S1
    xt = inputs.T
    params = pltpu.CompilerParams(dimension_semantics=("arbitrary",))

    Cb = 8
    unary_t = pl.pallas_call(
        functools.partial(_unary_kernel, L=L, Cb=Cb),
        grid=(pl.cdiv(C, Cb),),
        in_specs=[pl.BlockSpec((L, B), lambda c: (0, 0))],
        out_specs=pl.BlockSpec((Cb, L, B), lambda c: (c, 0, 0)),
        out_shape=jax.ShapeDtypeStruct((C, L, B), jnp.float32),
        compiler_params=params,
    )(xt)

    Ib = 8
    binary_t = pl.pallas_call(
        functools.partial(_binary_kernel, L=L, Ib=Ib),
        grid=(pl.cdiv(L, Ib),),
        in_specs=[
            pl.BlockSpec((Ib, B), lambda i: (i, 0)),
            pl.BlockSpec((L, B), lambda i: (0, 0)),
        ],
        out_specs=pl.BlockSpec((Ib, L - 1, B), lambda i: (i, 0, 0)),
        out_shape=jax.ShapeDtypeStruct((L, L - 1, B), jnp.float32),
        compiler_params=params,
    )(xt, xt)

    unary = jnp.transpose(unary_t, (2, 1, 0))
    binary = jnp.transpose(binary_t, (2, 0, 1))[..., None]
    return unary, binary


def kernel(inputs, table):
    del table
    return _run(inputs)


# speedup vs baseline: 4.7573x; 2.0330x over previous
import functools

import jax
import jax.numpy as jnp
from jax.experimental import pallas as pl
from jax.experimental.pallas import tpu as pltpu


def _unary_kernel(xt_ref, out_ref, *, L, Cb):
    c0 = pl.program_id(0) * Cb
    xt = xt_ref[...]
    Bb = xt.shape[1]
    ci = jax.lax.broadcasted_iota(jnp.int32, (Cb, L, Bb), 0) + c0
    ii = jax.lax.broadcasted_iota(jnp.int32, (Cb, L, Bb), 1)
    out_ref[...] = ((ci == ii) | (ci - L == xt[None, :, :])).astype(jnp.float32)


def _binary_kernel(xi_ref, xt_ref, out_ref, *, L, Ib):
    i0 = pl.program_id(0) * Ib
    xi = xi_ref[...]
    xt = xt_ref[...]
    U, V = xt.shape[1], xt.shape[2]
    shape = (Ib, L - 1, U, V)
    kk = jax.lax.broadcasted_iota(jnp.int32, shape, 1)
    ii = jax.lax.broadcasted_iota(jnp.int32, shape, 0) + i0
    xk0 = xt[None, : L - 1, :, :]
    xk1 = xt[None, 1:L, :, :]
    xj = jnp.where(kk < ii, xk0, xk1)
    out_ref[...] = (xi[:, None, :, :] == xj).astype(jnp.float32)


@jax.jit
def _run(inputs):
    B, L = inputs.shape
    S1 = 101
    C = L + S1
    xt = inputs.T
    params = pltpu.CompilerParams(dimension_semantics=("arbitrary",))

    Cb = 8
    unary_t = pl.pallas_call(
        functools.partial(_unary_kernel, L=L, Cb=Cb),
        grid=(pl.cdiv(C, Cb),),
        in_specs=[pl.BlockSpec((L, B), lambda c: (0, 0))],
        out_specs=pl.BlockSpec((Cb, L, B), lambda c: (c, 0, 0)),
        out_shape=jax.ShapeDtypeStruct((C, L, B), jnp.float32),
        compiler_params=params,
    )(xt)

    Ib = 8
    U, V = B // 128, 128
    xt4 = xt.reshape(L, U, V)
    binary_t = pl.pallas_call(
        functools.partial(_binary_kernel, L=L, Ib=Ib),
        grid=(pl.cdiv(L, Ib),),
        in_specs=[
            pl.BlockSpec((Ib, U, V), lambda i: (i, 0, 0)),
            pl.BlockSpec((L, U, V), lambda i: (0, 0, 0)),
        ],
        out_specs=pl.BlockSpec((Ib, L - 1, U, V), lambda i: (i, 0, 0, 0)),
        out_shape=jax.ShapeDtypeStruct((L, L - 1, U, V), jnp.float32),
        compiler_params=params,
    )(xt4, xt4)

    unary = jnp.transpose(unary_t, (2, 1, 0))
    binary = (
        jnp.transpose(binary_t, (2, 3, 0, 1)).reshape(B, L, L - 1)[..., None]
    )
    return unary, binary


def kernel(inputs, table):
    del table
    return _run(inputs)
